# baseline (speedup 1.0000x reference)
import jax
import jax.numpy as jnp
from jax.experimental import pallas as pl

_ACTION_DIM = 7


def _sanitize(x):
    return jnp.where(jnp.isfinite(x), x, 0.0)


def _mask_kernel(cols_ref, out_ref):
    n = out_ref.shape[1]
    x = cols_ref[...]
    p0 = _sanitize(x[0:n])
    p4 = _sanitize(x[n:2 * n])
    ex = _sanitize(x[2 * n:3 * n])

    has = p0 > 0.5
    hx = ex >= 0.9
    asl = p4 >= 0.9

    all_has = jnp.min(p0) > 0.5

    not_hx = jnp.logical_not(hx)
    buy = jnp.logical_not(has) & not_hx
    c6 = has & not_hx & jnp.logical_not(jnp.logical_and(all_has, asl))

    buy8 = buy.astype(jnp.int8).reshape(1, n)
    has8 = has.astype(jnp.int8).reshape(1, n)
    c68 = c6.astype(jnp.int8).reshape(1, n)

    out_ref[0:1, :] = jnp.ones((1, n), dtype=jnp.int8)
    out_ref[1:2, :] = buy8
    out_ref[2:3, :] = buy8
    out_ref[3:4, :] = buy8
    out_ref[4:5, :] = has8
    out_ref[5:6, :] = has8
    out_ref[6:7, :] = c68


@jax.jit
def kernel(position, portfolio):
    position = position.astype(jnp.float32)
    portfolio = portfolio.astype(jnp.float32)
    batch = position.shape[0]
    cols = jnp.concatenate(
        [position[:, 0], position[:, 4], portfolio[:, 2]], axis=0
    )
    raw = pl.pallas_call(
        _mask_kernel,
        out_shape=jax.ShapeDtypeStruct((_ACTION_DIM, batch), jnp.int8),
    )(cols)
    return raw.T.astype(jnp.bool_)
